# baseline (speedup 1.0000x reference)
import jax
import jax.numpy as jnp
from jax.experimental import pallas as pl
from jax.experimental.pallas import tpu as pltpu


def _gcn_body(x_ref, adj_ref, w1_ref, b1_ref, w2_ref, b2_ref, o_ref,
              s1_ref, s2_ref):
    p = pl.program_id(0)
    i = pl.program_id(1)

    @pl.when(jnp.logical_and(p == 0, i == 0))
    def _():
        s1_ref[...] = jnp.dot(x_ref[...], w1_ref[...],
                              preferred_element_type=jnp.float32)

    nblk = pl.num_programs(1)

    @pl.when(p == 0)
    def _():
        t = jnp.dot(adj_ref[...], s1_ref[...],
                    preferred_element_type=jnp.float32) + b1_ref[...]
        t = jnp.where(t >= 0, t, 0.01 * t)
        bm = adj_ref.shape[0]
        s2_ref[pl.ds(i * bm, bm), :] = jnp.dot(
            t, w2_ref[...], preferred_element_type=jnp.float32)
        o_ref[...] = jnp.zeros_like(o_ref)

    @pl.when(p == 1)
    def _():
        o_ref[...] = jnp.dot(adj_ref[...], s2_ref[...],
                             preferred_element_type=jnp.float32) + b2_ref[...]


def kernel(x, adj, W1, b1, W2, b2):
    n, nfeat = x.shape
    nhid = W1.shape[1]
    bm = 200

    b1r = b1.reshape(1, nhid)
    b2r = b2.reshape(1, nhid)
    grid = (2, n // bm)

    out = pl.pallas_call(
        _gcn_body,
        grid=grid,
        in_specs=[
            pl.BlockSpec((n, nfeat), lambda p, i: (0, 0)),
            pl.BlockSpec((bm, n), lambda p, i: (i, 0)),
            pl.BlockSpec((nfeat, nhid), lambda p, i: (0, 0)),
            pl.BlockSpec((1, nhid), lambda p, i: (0, 0)),
            pl.BlockSpec((nhid, nhid), lambda p, i: (0, 0)),
            pl.BlockSpec((1, nhid), lambda p, i: (0, 0)),
        ],
        out_specs=pl.BlockSpec((bm, nhid), lambda p, i: (i, 0)),
        out_shape=jax.ShapeDtypeStruct((n, nhid), jnp.float32),
        scratch_shapes=[
            pltpu.VMEM((n, nhid), jnp.float32),
            pltpu.VMEM((n, nhid), jnp.float32),
        ],
    )(x, adj, W1, b1r, W2, b2r)
    return out


# speedup vs baseline: 1.0053x; 1.0053x over previous
import functools

import jax
import jax.numpy as jnp
from jax.experimental import pallas as pl
from jax.experimental.pallas import tpu as pltpu

_NBUF = 3


def _s1t_body(x_ref, w1_ref, o_ref):
    o_ref[...] = jax.lax.dot_general(
        w1_ref[...], x_ref[...], (((0,), (1,)), ((), ())),
        preferred_element_type=jnp.float32)


def _gcn_body(s1t_ref, adj_hbm, b1_ref, w2_ref, b2_ref, o_ref,
              buf, s2_ref, sem, *, bm):
    n = s1t_ref.shape[1]
    nstrips = n // bm

    def copy(s):
        phys = jax.lax.rem(s, nstrips)
        slot = jax.lax.rem(s, _NBUF)
        return pltpu.make_async_copy(
            adj_hbm.at[pl.ds(phys * bm, bm), :],
            buf.at[slot],
            sem.at[slot],
        )

    for s in range(_NBUF):
        copy(s).start()

    def pass1_step(s, _):
        slot = jax.lax.rem(s, _NBUF)
        copy(s).wait()
        t = jax.lax.dot_general(
            buf[slot], s1t_ref[...], (((1,), (1,)), ((), ())),
            preferred_element_type=jnp.float32) + b1_ref[...]
        t = jnp.where(t >= 0, t, 0.01 * t)
        s2_ref[pl.ds(s * bm, bm), :] = jnp.dot(
            t, w2_ref[...], preferred_element_type=jnp.float32)
        copy(s + _NBUF).start()
        return 0

    def pass2_step(s, _):
        slot = jax.lax.rem(s, _NBUF)
        copy(s).wait()
        phys = s - nstrips
        o_ref[pl.ds(phys * bm, bm), :] = jnp.dot(
            buf[slot], s2_ref[...],
            preferred_element_type=jnp.float32) + b2_ref[...]

        @pl.when(s + _NBUF < 2 * nstrips)
        def _():
            copy(s + _NBUF).start()
        return 0

    jax.lax.fori_loop(0, nstrips, pass1_step, 0)
    jax.lax.fori_loop(nstrips, 2 * nstrips, pass2_step, 0)


def kernel(x, adj, W1, b1, W2, b2):
    n, nfeat = x.shape
    nhid = W1.shape[1]
    bm = 400

    s1t = pl.pallas_call(
        _s1t_body,
        out_shape=jax.ShapeDtypeStruct((nhid, n), jnp.float32),
    )(x, W1)

    b1r = b1.reshape(1, nhid)
    b2r = b2.reshape(1, nhid)

    out = pl.pallas_call(
        functools.partial(_gcn_body, bm=bm),
        in_specs=[
            pl.BlockSpec(memory_space=pltpu.MemorySpace.VMEM),
            pl.BlockSpec(memory_space=pl.ANY),
            pl.BlockSpec(memory_space=pltpu.MemorySpace.VMEM),
            pl.BlockSpec(memory_space=pltpu.MemorySpace.VMEM),
            pl.BlockSpec(memory_space=pltpu.MemorySpace.VMEM),
        ],
        out_shape=jax.ShapeDtypeStruct((n, nhid), jnp.float32),
        scratch_shapes=[
            pltpu.VMEM((_NBUF, bm, n), jnp.float32),
            pltpu.VMEM((n, nhid), jnp.float32),
            pltpu.SemaphoreType.DMA((_NBUF,)),
        ],
    )(s1t, adj, b1r, W2, b2r)
    return out
